# baseline (speedup 1.0000x reference)
import functools
import math

import jax
import jax.numpy as jnp
from jax.experimental import pallas as pl
from jax.experimental.pallas import tpu as pltpu

_K = 64
_CB = 4


def _attn_manual_kernel(x_hbm, o_hbm, xbuf, obuf, lsem, ssem, *, scale, nchunks):
    pltpu.make_async_copy(x_hbm.at[0], xbuf.at[0], lsem.at[0]).start()

    def step(c, _):
        slot = jax.lax.rem(c, 2)
        nxt = jax.lax.rem(c + 1, 2)

        @pl.when(c + 1 < nchunks)
        def _start_next_load():
            pltpu.make_async_copy(x_hbm.at[c + 1], xbuf.at[nxt], lsem.at[nxt]).start()

        pltpu.make_async_copy(x_hbm.at[c], xbuf.at[slot], lsem.at[slot]).wait()

        xb = xbuf[slot]
        xh = xb.astype(jnp.bfloat16)
        s = jax.lax.dot_general(
            xh, xh, (((2,), (2,)), ((0,), (0,))),
            preferred_element_type=jnp.float32) * scale
        n = xb.shape[1]
        row = jax.lax.broadcasted_iota(jnp.int32, (1, n, n), 1)
        col = jax.lax.broadcasted_iota(jnp.int32, (1, n, n), 2)
        s = jnp.where(col <= row, s, -jnp.inf)
        m = jnp.max(s, axis=2, keepdims=True)
        p = jnp.exp(s - m)
        z = jnp.sum(p, axis=2, keepdims=True)
        p = (p / z).astype(jnp.bfloat16)
        o = jax.lax.dot_general(
            p, xh, (((2,), (1,)), ((0,), (0,))),
            preferred_element_type=jnp.float32)

        @pl.when(c >= 2)
        def _wait_prev_store():
            pltpu.make_async_copy(obuf.at[slot], o_hbm.at[c - 2], ssem.at[slot]).wait()

        obuf[slot] = o
        pltpu.make_async_copy(obuf.at[slot], o_hbm.at[c], ssem.at[slot]).start()
        return 0

    jax.lax.fori_loop(0, nchunks, step, 0)
    last = nchunks - 1
    pltpu.make_async_copy(obuf.at[jax.lax.rem(last, 2)], o_hbm.at[last],
                          ssem.at[jax.lax.rem(last, 2)]).wait()

    @pl.when(nchunks >= 2)
    def _drain_prev():
        pltpu.make_async_copy(obuf.at[jax.lax.rem(last - 1, 2)], o_hbm.at[last - 1],
                              ssem.at[jax.lax.rem(last - 1, 2)]).wait()


def kernel(x):
    b, t, e = x.shape
    scale = 1.0 / math.sqrt(e)
    nblocks = t // _K
    nchunks = nblocks // _CB
    x2 = x.reshape(nchunks, _CB, _K, e)
    out = pl.pallas_call(
        functools.partial(_attn_manual_kernel, scale=scale, nchunks=nchunks),
        in_specs=[pl.BlockSpec(memory_space=pl.ANY)],
        out_specs=pl.BlockSpec(memory_space=pl.ANY),
        out_shape=jax.ShapeDtypeStruct((nchunks, _CB, _K, e), jnp.float32),
        scratch_shapes=[
            pltpu.VMEM((2, _CB, _K, e), jnp.float32),
            pltpu.VMEM((2, _CB, _K, e), jnp.float32),
            pltpu.SemaphoreType.DMA((2,)),
            pltpu.SemaphoreType.DMA((2,)),
        ],
    )(x2)
    return out.reshape(b, t, e)


# speedup vs baseline: 1.1052x; 1.1052x over previous
import functools
import math

import jax
import jax.numpy as jnp
from jax.experimental import pallas as pl
from jax.experimental.pallas import tpu as pltpu

_K = 64
_CB = 4


def _attn_manual_kernel(x_hbm, o_hbm, xbuf, obuf, lsem, ssem, *, scale, nchunks):
    def load(c):
        return pltpu.make_async_copy(x_hbm.at[c], xbuf.at[c % 2], lsem.at[c % 2])

    def store(c):
        return pltpu.make_async_copy(obuf.at[c % 2], o_hbm.at[c], ssem.at[c % 2])

    load(0).start()
    for c in range(nchunks):
        if c + 1 < nchunks:
            load(c + 1).start()
        load(c).wait()
        xh = xbuf[c % 2].astype(jnp.bfloat16)
        s = jax.lax.dot_general(
            xh, xh, (((2,), (2,)), ((0,), (0,))),
            preferred_element_type=jnp.float32) * scale
        n = xh.shape[1]
        row = jax.lax.broadcasted_iota(jnp.int32, (1, n, n), 1)
        col = jax.lax.broadcasted_iota(jnp.int32, (1, n, n), 2)
        s = jnp.where(col <= row, s, -jnp.inf)
        m = jnp.max(s, axis=2, keepdims=True)
        p = jnp.exp(s - m)
        z = jnp.sum(p, axis=2, keepdims=True)
        p = (p / z).astype(jnp.bfloat16)
        o = jax.lax.dot_general(
            p, xh, (((2,), (1,)), ((0,), (0,))),
            preferred_element_type=jnp.float32)
        if c >= 2:
            store(c - 2).wait()
        obuf[c % 2] = o
        store(c).start()
    for c in range(max(0, nchunks - 2), nchunks):
        store(c).wait()


def kernel(x):
    b, t, e = x.shape
    scale = 1.0 / math.sqrt(e)
    nblocks = t // _K
    nchunks = nblocks // _CB
    x2 = x.reshape(nchunks, _CB, _K, e)
    out = pl.pallas_call(
        functools.partial(_attn_manual_kernel, scale=scale, nchunks=nchunks),
        in_specs=[pl.BlockSpec(memory_space=pl.ANY)],
        out_specs=pl.BlockSpec(memory_space=pl.ANY),
        out_shape=jax.ShapeDtypeStruct((nchunks, _CB, _K, e), jnp.float32),
        scratch_shapes=[
            pltpu.VMEM((2, _CB, _K, e), jnp.float32),
            pltpu.VMEM((2, _CB, _K, e), jnp.float32),
            pltpu.SemaphoreType.DMA((2,)),
            pltpu.SemaphoreType.DMA((2,)),
        ],
    )(x2)
    return out.reshape(b, t, e)


# speedup vs baseline: 1.4319x; 1.2957x over previous
import functools
import math

import jax
import jax.numpy as jnp
from jax.experimental import pallas as pl
from jax.experimental.pallas import tpu as pltpu

_K = 64
_CB = 8


def _attn_manual_kernel(x_hbm, o_hbm, xbuf, obuf, lsem, ssem, *, scale, nchunks):
    def load(c):
        return pltpu.make_async_copy(x_hbm.at[c], xbuf.at[c % 2], lsem.at[c % 2])

    def store(c):
        return pltpu.make_async_copy(obuf.at[c % 2], o_hbm.at[c], ssem.at[c % 2])

    load(0).start()
    for c in range(nchunks):
        if c + 1 < nchunks:
            load(c + 1).start()
        load(c).wait()
        xh = xbuf[c % 2].astype(jnp.bfloat16)
        s = jax.lax.dot_general(
            xh, xh, (((2,), (2,)), ((0,), (0,))),
            preferred_element_type=jnp.float32) * scale
        n = xh.shape[1]
        row = jax.lax.broadcasted_iota(jnp.int32, (1, n, n), 1)
        col = jax.lax.broadcasted_iota(jnp.int32, (1, n, n), 2)
        s = jnp.where(col <= row, s, -jnp.inf)
        m = jnp.max(s, axis=2, keepdims=True)
        p = jnp.exp(s - m)
        z = jnp.sum(p, axis=2, keepdims=True)
        p = (p / z).astype(jnp.bfloat16)
        o = jax.lax.dot_general(
            p, xh, (((2,), (1,)), ((0,), (0,))),
            preferred_element_type=jnp.float32)
        if c >= 2:
            store(c - 2).wait()
        obuf[c % 2] = o
        store(c).start()
    for c in range(max(0, nchunks - 2), nchunks):
        store(c).wait()


def kernel(x):
    b, t, e = x.shape
    scale = 1.0 / math.sqrt(e)
    nblocks = t // _K
    nchunks = nblocks // _CB
    x2 = x.reshape(nchunks, _CB, _K, e)
    out = pl.pallas_call(
        functools.partial(_attn_manual_kernel, scale=scale, nchunks=nchunks),
        in_specs=[pl.BlockSpec(memory_space=pl.ANY)],
        out_specs=pl.BlockSpec(memory_space=pl.ANY),
        out_shape=jax.ShapeDtypeStruct((nchunks, _CB, _K, e), jnp.float32),
        scratch_shapes=[
            pltpu.VMEM((2, _CB, _K, e), jnp.float32),
            pltpu.VMEM((2, _CB, _K, e), jnp.float32),
            pltpu.SemaphoreType.DMA((2,)),
            pltpu.SemaphoreType.DMA((2,)),
        ],
    )(x2)
    return out.reshape(b, t, e)


# speedup vs baseline: 1.6815x; 1.1743x over previous
import functools
import math

import jax
import jax.numpy as jnp
from jax.experimental import pallas as pl
from jax.experimental.pallas import tpu as pltpu

_K = 64
_CB = 8
_NC = 4


def _attn_manual_kernel(x_hbm, o_hbm, xbuf, obuf, lsem, ssem, *, scale):
    def load(c):
        return pltpu.make_async_copy(x_hbm.at[c], xbuf.at[c], lsem.at[c])

    def store(c):
        return pltpu.make_async_copy(obuf.at[c], o_hbm.at[c], ssem.at[c])

    for c in range(_NC):
        load(c).start()
    for c in range(_NC):
        load(c).wait()
        xh = xbuf[c].astype(jnp.bfloat16)
        s = jax.lax.dot_general(
            xh, xh, (((2,), (2,)), ((0,), (0,))),
            preferred_element_type=jnp.float32) * scale
        n = xh.shape[1]
        row = jax.lax.broadcasted_iota(jnp.int32, (1, n, n), 1)
        col = jax.lax.broadcasted_iota(jnp.int32, (1, n, n), 2)
        s = jnp.where(col <= row, s, -jnp.inf)
        m = jnp.max(s, axis=2, keepdims=True)
        p = jnp.exp(s - m)
        z = jnp.sum(p, axis=2, keepdims=True)
        p = (p / z).astype(jnp.bfloat16)
        obuf[c] = jax.lax.dot_general(
            p, xh, (((2,), (1,)), ((0,), (0,))),
            preferred_element_type=jnp.float32)
        store(c).start()
    for c in range(_NC):
        store(c).wait()


def kernel(x):
    b, t, e = x.shape
    scale = 1.0 / math.sqrt(e)
    nblocks = t // _K
    assert nblocks == _CB * _NC
    x2 = x.reshape(_NC, _CB, _K, e)
    out = pl.pallas_call(
        functools.partial(_attn_manual_kernel, scale=scale),
        in_specs=[pl.BlockSpec(memory_space=pl.ANY)],
        out_specs=pl.BlockSpec(memory_space=pl.ANY),
        out_shape=jax.ShapeDtypeStruct((_NC, _CB, _K, e), jnp.float32),
        scratch_shapes=[
            pltpu.VMEM((_NC, _CB, _K, e), jnp.float32),
            pltpu.VMEM((_NC, _CB, _K, e), jnp.float32),
            pltpu.SemaphoreType.DMA((_NC,)),
            pltpu.SemaphoreType.DMA((_NC,)),
        ],
    )(x2)
    return out.reshape(b, t, e)
